# Initial kernel scaffold; baseline (speedup 1.0000x reference)
import jax
import jax.numpy as jnp
from jax.experimental import pallas as pl


def kernel(logits, mask, noise):
    raise NotImplementedError("write your pallas kernel here")


# speedup vs baseline: 1.5479x; 1.5479x over previous
import jax
import jax.numpy as jnp
from jax.experimental import pallas as pl

B, N = 8192, 4096
ROWS = 256


def _body(logits_ref, mask_ref, noise_ref, action_ref, lp_ref):
    x = logits_ref[...]
    m = mask_ref[...]
    u = noise_ref[...]

    neg_inf = jnp.float32(-jnp.inf)
    masked = jnp.where(m, x, neg_inf)

    row_max = jnp.max(masked, axis=1, keepdims=True)
    s = jnp.sum(jnp.where(m, jnp.exp(masked - row_max), 0.0), axis=1)

    gumbel = -jnp.log(-jnp.log(u))
    score = masked + gumbel
    best = jnp.max(score, axis=1, keepdims=True)

    idx = jax.lax.broadcasted_iota(jnp.int32, score.shape, 1)
    action = jnp.min(jnp.where(score == best, idx, N), axis=1)

    val = jnp.max(jnp.where(idx == action[:, None], masked, neg_inf), axis=1)
    log_prob = val - (row_max[:, 0] + jnp.log(s))

    action_ref[...] = action[:, None]
    lp_ref[...] = log_prob[:, None]


def kernel(logits, mask, noise):
    grid = (B // ROWS,)
    action2d, lp2d = pl.pallas_call(
        _body,
        grid=grid,
        in_specs=[
            pl.BlockSpec((ROWS, N), lambda i: (i, 0)),
            pl.BlockSpec((ROWS, N), lambda i: (i, 0)),
            pl.BlockSpec((ROWS, N), lambda i: (i, 0)),
        ],
        out_specs=[
            pl.BlockSpec((ROWS, 1), lambda i: (i, 0)),
            pl.BlockSpec((ROWS, 1), lambda i: (i, 0)),
        ],
        out_shape=[
            jax.ShapeDtypeStruct((B, 1), jnp.int32),
            jax.ShapeDtypeStruct((B, 1), jnp.float32),
        ],
    )(logits, mask, noise)
    return (action2d[:, 0], lp2d[:, 0])


# speedup vs baseline: 1.5640x; 1.0104x over previous
import jax
import jax.numpy as jnp
from jax.experimental import pallas as pl

B, N = 8192, 4096
ROWS = 256


def _body(logits_ref, mask_ref, noise_ref, action_ref, lp_ref):
    x = logits_ref[...]
    m = mask_ref[...]
    u = noise_ref[...]

    neg_inf = jnp.float32(-jnp.inf)
    masked = jnp.where(m, x, neg_inf)

    row_max = jnp.max(masked, axis=1, keepdims=True)
    s = jnp.sum(jnp.exp(masked - row_max), axis=1)

    gumbel = -jnp.log(-jnp.log(u))
    score = masked + gumbel
    best = jnp.max(score, axis=1, keepdims=True)

    eq = score == best
    idx = jax.lax.broadcasted_iota(jnp.int32, (1, N), 1)
    action = jnp.min(jnp.where(eq, idx, N), axis=1)

    val = jnp.max(jnp.where(eq, masked, neg_inf), axis=1)
    log_prob = val - (row_max[:, 0] + jnp.log(s))

    action_ref[...] = action[:, None]
    lp_ref[...] = log_prob[:, None]


def kernel(logits, mask, noise):
    grid = (B // ROWS,)
    action2d, lp2d = pl.pallas_call(
        _body,
        grid=grid,
        in_specs=[
            pl.BlockSpec((ROWS, N), lambda i: (i, 0)),
            pl.BlockSpec((ROWS, N), lambda i: (i, 0)),
            pl.BlockSpec((ROWS, N), lambda i: (i, 0)),
        ],
        out_specs=[
            pl.BlockSpec((ROWS, 1), lambda i: (i, 0)),
            pl.BlockSpec((ROWS, 1), lambda i: (i, 0)),
        ],
        out_shape=[
            jax.ShapeDtypeStruct((B, 1), jnp.int32),
            jax.ShapeDtypeStruct((B, 1), jnp.float32),
        ],
    )(logits, mask, noise)
    return (action2d[:, 0], lp2d[:, 0])
